# initial kernel scaffold (unmeasured)
import jax
import jax.numpy as jnp
from jax import lax
from jax.experimental import pallas as pl
from jax.experimental.pallas import tpu as pltpu

N_DEV = 4
N_CHUNK = 2048


def kernel(x, w_mat):
    k_tot, k_blk = x.shape
    m_blk = k_tot // N_DEV
    n_tot = w_mat.shape[1]
    n_chunks = n_tot // N_CHUNK
    hop_order = (1, 3, 2)

    def body(x_ref, w_ref, out_ref, x_bf, recv_buf, w_buf,
             send_sems, recv_sems, w_sems):
        my = lax.axis_index("i")

        barrier = pltpu.get_barrier_semaphore()
        for d in range(1, N_DEV):
            pl.semaphore_signal(
                barrier, inc=1,
                device_id=((my + d) % N_DEV,),
                device_id_type=pl.DeviceIdType.MESH,
            )
        pl.semaphore_wait(barrier, N_DEV - 1)

        x_bf[...] = x_ref[...].astype(jnp.bfloat16)

        rdmas = []
        for d in range(1, N_DEV):
            dst = (my + d) % N_DEV
            rdma = pltpu.make_async_remote_copy(
                src_ref=x_bf.at[pl.ds(dst * m_blk, m_blk), :],
                dst_ref=recv_buf.at[d - 1],
                send_sem=send_sems.at[d - 1],
                recv_sem=recv_sems.at[d - 1],
                device_id=(dst,),
                device_id_type=pl.DeviceIdType.MESH,
            )
            rdma.start()
            rdmas.append(rdma)

        def src_k(p):
            if p == 0:
                return my
            return (my + (N_DEV - hop_order[p - 1])) % N_DEV

        def issue_w(step, slot):
            p, c = step // n_chunks, step % n_chunks
            cp = pltpu.make_async_copy(
                w_ref.at[pl.ds(src_k(p) * k_blk, k_blk),
                         pl.ds(c * N_CHUNK, N_CHUNK)],
                w_buf.at[slot],
                w_sems.at[slot],
            )
            cp.start()
            return cp

        n_steps = N_DEV * n_chunks
        copies = [None] * n_steps
        copies[0] = issue_w(0, 0)
        for step in range(n_steps):
            p, c = step // n_chunks, step % n_chunks
            slot = step % 2
            if step + 1 < n_steps:
                copies[step + 1] = issue_w(step + 1, (step + 1) % 2)
            if p > 0 and c == 0:
                rdmas[hop_order[p - 1] - 1].wait_recv()
            if p == 0:
                a = x_ref[pl.ds(my * m_blk, m_blk), :]
            else:
                a = recv_buf[hop_order[p - 1] - 1].astype(jnp.float32)
            copies[step].wait()
            part = jnp.dot(a, w_buf[slot], preferred_element_type=jnp.float32)
            if p == 0:
                out_ref[:, pl.ds(c * N_CHUNK, N_CHUNK)] = part
            else:
                out_ref[:, pl.ds(c * N_CHUNK, N_CHUNK)] += part

        for rdma in rdmas:
            rdma.wait_send()

    return pl.pallas_call(
        body,
        out_shape=jax.ShapeDtypeStruct((m_blk, n_tot), jnp.float32),
        in_specs=[
            pl.BlockSpec(memory_space=pltpu.VMEM),
            pl.BlockSpec(memory_space=pltpu.ANY),
        ],
        out_specs=pl.BlockSpec(memory_space=pltpu.VMEM),
        scratch_shapes=[
            pltpu.VMEM((k_tot, k_blk), jnp.bfloat16),
            pltpu.VMEM((N_DEV - 1, m_blk, k_blk), jnp.bfloat16),
            pltpu.VMEM((2, k_blk, N_CHUNK), jnp.float32),
            pltpu.SemaphoreType.DMA((N_DEV - 1,)),
            pltpu.SemaphoreType.DMA((N_DEV - 1,)),
            pltpu.SemaphoreType.DMA((2,)),
        ],
        compiler_params=pltpu.CompilerParams(collective_id=0),
    )(x, w_mat)


# baseline (device time: 150025 ns/iter reference)
import jax
import jax.numpy as jnp
from jax import lax
from jax.experimental import pallas as pl
from jax.experimental.pallas import tpu as pltpu

N_DEV = 4
N_CHUNK = 512


def kernel(x, w_mat):
    k_tot, k_blk = x.shape
    m_blk = k_tot // N_DEV
    n_tot = w_mat.shape[1]
    n_chunks = n_tot // N_CHUNK
    hop_order = (1, 3, 2)

    def body(x_ref, w_ref, out_ref, stage, x_send, recv_buf, w_buf,
             send_sems, recv_sems, w_sems, stage_sems):
        my = lax.axis_index("i")

        barrier = pltpu.get_barrier_semaphore()
        for d in range(1, N_DEV):
            pl.semaphore_signal(
                barrier, inc=1,
                device_id=((my + d) % N_DEV,),
                device_id_type=pl.DeviceIdType.MESH,
            )
        pl.semaphore_wait(barrier, N_DEV - 1)

        def stage_in(row_block, slot):
            cp = pltpu.make_async_copy(
                x_ref.at[pl.ds(row_block * m_blk, m_blk), :],
                stage.at[slot],
                stage_sems.at[slot],
            )
            cp.start()
            return cp

        local_cp = stage_in(my, 0)
        rdmas = []
        pending = stage_in((my + 1) % N_DEV, 1)
        for d in range(1, N_DEV):
            pending.wait()
            x_send[d - 1] = stage[1].astype(jnp.bfloat16)
            if d < N_DEV - 1:
                pending = stage_in((my + d + 1) % N_DEV, 1)
            dst = (my + d) % N_DEV
            rdma = pltpu.make_async_remote_copy(
                src_ref=x_send.at[d - 1],
                dst_ref=recv_buf.at[d - 1],
                send_sem=send_sems.at[d - 1],
                recv_sem=recv_sems.at[d - 1],
                device_id=(dst,),
                device_id_type=pl.DeviceIdType.MESH,
            )
            rdma.start()
            rdmas.append(rdma)

        def src_k(p):
            if p == 0:
                return my
            return (my + (N_DEV - hop_order[p - 1])) % N_DEV

        def issue_w(step, slot):
            p, c = step // n_chunks, step % n_chunks
            cp = pltpu.make_async_copy(
                w_ref.at[pl.ds(src_k(p) * k_blk, k_blk),
                         pl.ds(c * N_CHUNK, N_CHUNK)],
                w_buf.at[slot],
                w_sems.at[slot],
            )
            cp.start()
            return cp

        n_steps = N_DEV * n_chunks
        copies = [None] * n_steps
        copies[0] = issue_w(0, 0)
        local_cp.wait()
        for step in range(n_steps):
            p, c = step // n_chunks, step % n_chunks
            slot = step % 2
            if step + 1 < n_steps:
                copies[step + 1] = issue_w(step + 1, (step + 1) % 2)
            if p > 0 and c == 0:
                rdmas[hop_order[p - 1] - 1].wait_recv()
            if p == 0:
                a = stage[0]
            else:
                a = recv_buf[hop_order[p - 1] - 1].astype(jnp.float32)
            copies[step].wait()
            part = jnp.dot(a, w_buf[slot], preferred_element_type=jnp.float32)
            if p == 0:
                out_ref[:, pl.ds(c * N_CHUNK, N_CHUNK)] = part
            else:
                out_ref[:, pl.ds(c * N_CHUNK, N_CHUNK)] += part

        for rdma in rdmas:
            rdma.wait_send()

    return pl.pallas_call(
        body,
        out_shape=jax.ShapeDtypeStruct((m_blk, n_tot), jnp.float32),
        in_specs=[
            pl.BlockSpec(memory_space=pl.ANY),
            pl.BlockSpec(memory_space=pl.ANY),
        ],
        out_specs=pl.BlockSpec(memory_space=pltpu.VMEM),
        scratch_shapes=[
            pltpu.VMEM((2, m_blk, k_blk), jnp.float32),
            pltpu.VMEM((N_DEV - 1, m_blk, k_blk), jnp.bfloat16),
            pltpu.VMEM((N_DEV - 1, m_blk, k_blk), jnp.bfloat16),
            pltpu.VMEM((2, k_blk, N_CHUNK), jnp.float32),
            pltpu.SemaphoreType.DMA((N_DEV - 1,)),
            pltpu.SemaphoreType.DMA((N_DEV - 1,)),
            pltpu.SemaphoreType.DMA((2,)),
            pltpu.SemaphoreType.DMA((2,)),
        ],
        compiler_params=pltpu.CompilerParams(
            collective_id=0,
            vmem_limit_bytes=63 * 1024 * 1024,
        ),
    )(x, w_mat)


# device time: 146415 ns/iter; 1.0247x vs baseline; 1.0247x over previous
import jax
import jax.numpy as jnp
from jax import lax
from jax.experimental import pallas as pl
from jax.experimental.pallas import tpu as pltpu

N_DEV = 4
N_CHUNK = 512


def kernel(x, w_mat):
    k_tot, k_blk = x.shape
    m_blk = k_tot // N_DEV
    n_tot = w_mat.shape[1]
    n_chunks = n_tot // N_CHUNK
    hop_order = (1, 3, 2)

    def body(x_ref, w_ref, out_ref, stage, x_send, recv_buf, w_buf,
             a_buf, send_sems, recv_sems, w_sems, stage_sems):
        my = lax.axis_index("i")

        barrier = pltpu.get_barrier_semaphore()
        for d in range(1, N_DEV):
            pl.semaphore_signal(
                barrier, inc=1,
                device_id=((my + d) % N_DEV,),
                device_id_type=pl.DeviceIdType.MESH,
            )
        pl.semaphore_wait(barrier, N_DEV - 1)

        def stage_in(row_block, slot):
            cp = pltpu.make_async_copy(
                x_ref.at[pl.ds(row_block * m_blk, m_blk), :],
                stage.at[slot],
                stage_sems.at[slot],
            )
            cp.start()
            return cp

        local_cp = stage_in(my, 0)
        rdmas = []
        pending = stage_in((my + 1) % N_DEV, 1)
        for d in range(1, N_DEV):
            pending.wait()
            x_send[d - 1] = stage[1].astype(jnp.bfloat16)
            if d < N_DEV - 1:
                pending = stage_in((my + d + 1) % N_DEV, 1)
            dst = (my + d) % N_DEV
            rdma = pltpu.make_async_remote_copy(
                src_ref=x_send.at[d - 1],
                dst_ref=recv_buf.at[d - 1],
                send_sem=send_sems.at[d - 1],
                recv_sem=recv_sems.at[d - 1],
                device_id=(dst,),
                device_id_type=pl.DeviceIdType.MESH,
            )
            rdma.start()
            rdmas.append(rdma)

        def src_k(p):
            if p == 0:
                return my
            return (my + (N_DEV - hop_order[p - 1])) % N_DEV

        def issue_w(step, slot):
            p, c = step // n_chunks, step % n_chunks
            cp = pltpu.make_async_copy(
                w_ref.at[pl.ds(src_k(p) * k_blk, k_blk),
                         pl.ds(c * N_CHUNK, N_CHUNK)],
                w_buf.at[slot],
                w_sems.at[slot],
            )
            cp.start()
            return cp

        n_steps = N_DEV * n_chunks
        copies = [None] * n_steps
        copies[0] = issue_w(0, 0)
        local_cp.wait()
        a_buf[...] = stage[0].astype(jnp.bfloat16)
        for step in range(n_steps):
            p, c = step // n_chunks, step % n_chunks
            slot = step % 2
            if step + 1 < n_steps:
                copies[step + 1] = issue_w(step + 1, (step + 1) % 2)
            if p > 0 and c == 0:
                rdmas[hop_order[p - 1] - 1].wait_recv()
            if p == 0:
                a = a_buf[...]
            else:
                a = recv_buf[hop_order[p - 1] - 1]
            copies[step].wait()
            part = jnp.dot(a, w_buf[slot].astype(jnp.bfloat16),
                           preferred_element_type=jnp.float32)
            if p == 0:
                out_ref[:, pl.ds(c * N_CHUNK, N_CHUNK)] = part
            else:
                out_ref[:, pl.ds(c * N_CHUNK, N_CHUNK)] += part

        for rdma in rdmas:
            rdma.wait_send()

    return pl.pallas_call(
        body,
        out_shape=jax.ShapeDtypeStruct((m_blk, n_tot), jnp.float32),
        in_specs=[
            pl.BlockSpec(memory_space=pl.ANY),
            pl.BlockSpec(memory_space=pl.ANY),
        ],
        out_specs=pl.BlockSpec(memory_space=pltpu.VMEM),
        scratch_shapes=[
            pltpu.VMEM((2, m_blk, k_blk), jnp.float32),
            pltpu.VMEM((N_DEV - 1, m_blk, k_blk), jnp.bfloat16),
            pltpu.VMEM((N_DEV - 1, m_blk, k_blk), jnp.bfloat16),
            pltpu.VMEM((2, k_blk, N_CHUNK), jnp.float32),
            pltpu.VMEM((m_blk, k_blk), jnp.bfloat16),
            pltpu.SemaphoreType.DMA((N_DEV - 1,)),
            pltpu.SemaphoreType.DMA((N_DEV - 1,)),
            pltpu.SemaphoreType.DMA((2,)),
            pltpu.SemaphoreType.DMA((2,)),
        ],
        compiler_params=pltpu.CompilerParams(
            collective_id=0,
            vmem_limit_bytes=63 * 1024 * 1024,
        ),
    )(x, w_mat)


# device time: 125999 ns/iter; 1.1907x vs baseline; 1.1620x over previous
import jax
import jax.numpy as jnp
from jax import lax
from jax.experimental import pallas as pl
from jax.experimental.pallas import tpu as pltpu

N_DEV = 4
N_CHUNK = 1024
HOP_ORDER = (1, 3, 2)


def kernel(x, w_mat):
    k_tot, k_blk = x.shape
    m_blk = k_tot // N_DEV
    n_tot = w_mat.shape[1]
    n_chunks = n_tot // N_CHUNK

    def body(x_ref, w_ref, out_ref, stage, a_bf, x_send, recv_buf, w_buf,
             acc, send_sems, recv_sems, w_sems, stage_sem, out_sems):
        my = lax.axis_index("i")

        barrier = pltpu.get_barrier_semaphore()
        for d in range(1, N_DEV):
            pl.semaphore_signal(
                barrier, inc=1,
                device_id=((my + d) % N_DEV,),
                device_id_type=pl.DeviceIdType.MESH,
            )
        pl.semaphore_wait(barrier, N_DEV - 1)

        def stage_chunk(blk_idx):
            cp = pltpu.make_async_copy(
                x_ref.at[pl.ds(blk_idx * m_blk, m_blk), :], stage, stage_sem)
            cp.start()
            return cp

        def make_rdma(d):
            dst = (my + d) % N_DEV
            return pltpu.make_async_remote_copy(
                src_ref=x_send.at[d - 1],
                dst_ref=recv_buf.at[d - 1],
                send_sem=send_sems.at[d - 1],
                recv_sem=recv_sems.at[d - 1],
                device_id=(dst,),
                device_id_type=pl.DeviceIdType.MESH,
            )

        def src_k(p):
            if p == 0:
                return my
            return (my + (N_DEV - HOP_ORDER[p - 1])) % N_DEV

        def issue_w(step, slot):
            p, c = divmod(step, n_chunks)
            cp = pltpu.make_async_copy(
                w_ref.at[pl.ds(src_k(p) * k_blk, k_blk),
                         pl.ds(c * N_CHUNK, N_CHUNK)],
                w_buf.at[slot],
                w_sems.at[slot],
            )
            cp.start()
            return cp

        stage_chunk((my + 1) % N_DEV).wait()
        x_send[0] = stage[...].astype(jnp.bfloat16)
        rdma1 = make_rdma(1)
        rdma1.start()
        stage_chunk(my).wait()
        a_bf[...] = stage[...].astype(jnp.bfloat16)
        pending = stage_chunk((my + 3) % N_DEV)
        rdma3 = make_rdma(3)
        rdma2 = make_rdma(2)
        phase_rdma = {1: rdma1, 2: rdma3, 3: rdma2}

        n_steps = N_DEV * n_chunks
        copies = [None] * n_steps
        copies[0] = issue_w(0, 0)
        copies[1] = issue_w(1, 1)
        out_cps = []
        for step in range(n_steps):
            p, c = divmod(step, n_chunks)
            slot = step % 2
            if step == 1:
                pending.wait()
                x_send[2] = stage[...].astype(jnp.bfloat16)
                pending = stage_chunk((my + 2) % N_DEV)
            elif step == 3:
                pending.wait()
                x_send[1] = stage[...].astype(jnp.bfloat16)
            if step == 4:
                rdma3.start()
            elif step == 6:
                rdma2.start()
            if p > 0 and c == 0:
                phase_rdma[p].wait_recv()
            a = a_bf[...] if p == 0 else recv_buf[HOP_ORDER[p - 1] - 1]
            copies[step].wait()
            part = jnp.dot(a, w_buf[slot].astype(jnp.bfloat16),
                           preferred_element_type=jnp.float32)
            nds = pl.ds(c * N_CHUNK, N_CHUNK)
            if p == 0:
                acc[:, nds] = part
            else:
                acc[:, nds] += part
            if p == N_DEV - 1:
                if len(out_cps) >= 2:
                    out_cps[-2].wait()
                ocp = pltpu.make_async_copy(
                    acc.at[:, nds], out_ref.at[:, nds], out_sems.at[c % 2])
                ocp.start()
                out_cps.append(ocp)
            if step + 2 < n_steps:
                copies[step + 2] = issue_w(step + 2, slot)

        for ocp in out_cps[-2:]:
            ocp.wait()
        for rdma in (rdma1, rdma3, rdma2):
            rdma.wait_send()

    return pl.pallas_call(
        body,
        out_shape=jax.ShapeDtypeStruct((m_blk, n_tot), jnp.float32),
        in_specs=[
            pl.BlockSpec(memory_space=pl.ANY),
            pl.BlockSpec(memory_space=pl.ANY),
        ],
        out_specs=pl.BlockSpec(memory_space=pl.ANY),
        scratch_shapes=[
            pltpu.VMEM((m_blk, k_blk), jnp.float32),
            pltpu.VMEM((m_blk, k_blk), jnp.bfloat16),
            pltpu.VMEM((N_DEV - 1, m_blk, k_blk), jnp.bfloat16),
            pltpu.VMEM((N_DEV - 1, m_blk, k_blk), jnp.bfloat16),
            pltpu.VMEM((2, k_blk, N_CHUNK), jnp.float32),
            pltpu.VMEM((m_blk, n_tot), jnp.float32),
            pltpu.SemaphoreType.DMA((N_DEV - 1,)),
            pltpu.SemaphoreType.DMA((N_DEV - 1,)),
            pltpu.SemaphoreType.DMA((2,)),
            pltpu.SemaphoreType.DMA,
            pltpu.SemaphoreType.DMA((2,)),
        ],
        compiler_params=pltpu.CompilerParams(
            collective_id=0,
            vmem_limit_bytes=63 * 1024 * 1024,
        ),
    )(x, w_mat)


# device time: 125568 ns/iter; 1.1948x vs baseline; 1.0034x over previous
import jax
import jax.numpy as jnp
from jax import lax
from jax.experimental import pallas as pl
from jax.experimental.pallas import tpu as pltpu

N_DEV = 4
N_CHUNK = 1024
HOP_ORDER = (1, 3, 2)


def kernel(x, w_mat):
    k_tot, k_blk = x.shape
    m_blk = k_tot // N_DEV
    n_tot = w_mat.shape[1]
    n_chunks = n_tot // N_CHUNK

    def body(x_ref, w_ref, out_ref, stage, a_bf, x_send, recv_buf, w_buf,
             acc, send_sems, recv_sems, w_sems, stage_sem, out_sems):
        my = lax.axis_index("i")

        barrier = pltpu.get_barrier_semaphore()
        for d in range(1, N_DEV):
            pl.semaphore_signal(
                barrier, inc=1,
                device_id=((my + d) % N_DEV,),
                device_id_type=pl.DeviceIdType.MESH,
            )
        pl.semaphore_wait(barrier, N_DEV - 1)

        def stage_chunk(blk_idx):
            cp = pltpu.make_async_copy(
                x_ref.at[pl.ds(blk_idx * m_blk, m_blk), :], stage, stage_sem)
            cp.start()
            return cp

        def make_rdma(d):
            dst = (my + d) % N_DEV
            return pltpu.make_async_remote_copy(
                src_ref=x_send.at[d - 1],
                dst_ref=recv_buf.at[d - 1],
                send_sem=send_sems.at[d - 1],
                recv_sem=recv_sems.at[d - 1],
                device_id=(dst,),
                device_id_type=pl.DeviceIdType.MESH,
            )

        def src_k(p):
            if p == 0:
                return my
            return (my + (N_DEV - HOP_ORDER[p - 1])) % N_DEV

        def issue_w(step, slot):
            p, c = divmod(step, n_chunks)
            cp = pltpu.make_async_copy(
                w_ref.at[pl.ds(src_k(p) * k_blk, k_blk),
                         pl.ds(c * N_CHUNK, N_CHUNK)],
                w_buf.at[slot],
                w_sems.at[slot],
            )
            cp.start()
            return cp

        n_steps = N_DEV * n_chunks
        copies = [None] * n_steps
        copies[0] = issue_w(0, 0)
        copies[1] = issue_w(1, 1)

        stage_chunk((my + 1) % N_DEV).wait()
        x_send[0] = stage[...].astype(jnp.bfloat16)
        rdma1 = make_rdma(1)
        rdma1.start()
        stage_chunk(my).wait()
        a_bf[...] = stage[...].astype(jnp.bfloat16)
        pending = stage_chunk((my + 3) % N_DEV)
        rdma3 = make_rdma(3)
        rdma2 = make_rdma(2)
        phase_rdma = {1: rdma1, 2: rdma3, 3: rdma2}

        out_cps = []
        for step in range(n_steps):
            p, c = divmod(step, n_chunks)
            slot = step % 2
            if step == 1:
                pending.wait()
                x_send[2] = stage[...].astype(jnp.bfloat16)
                pending = stage_chunk((my + 2) % N_DEV)
            elif step == 3:
                pending.wait()
                x_send[1] = stage[...].astype(jnp.bfloat16)
            if step == 4:
                rdma3.start()
            elif step == 6:
                rdma2.start()
            if p > 0 and c == 0:
                phase_rdma[p].wait_recv()
            a = a_bf[...] if p == 0 else recv_buf[HOP_ORDER[p - 1] - 1]
            copies[step].wait()
            part = jnp.dot(a, w_buf[slot].astype(jnp.bfloat16),
                           preferred_element_type=jnp.float32)
            nds = pl.ds(c * N_CHUNK, N_CHUNK)
            if p == 0:
                acc[:, nds] = part
            else:
                acc[:, nds] += part
            if p == N_DEV - 1:
                if len(out_cps) >= 2:
                    out_cps[-2].wait()
                ocp = pltpu.make_async_copy(
                    acc.at[:, nds], out_ref.at[:, nds], out_sems.at[c % 2])
                ocp.start()
                out_cps.append(ocp)
            if step + 2 < n_steps:
                copies[step + 2] = issue_w(step + 2, slot)

        for ocp in out_cps[-2:]:
            ocp.wait()
        for rdma in (rdma1, rdma3, rdma2):
            rdma.wait_send()

    return pl.pallas_call(
        body,
        out_shape=jax.ShapeDtypeStruct((m_blk, n_tot), jnp.float32),
        in_specs=[
            pl.BlockSpec(memory_space=pl.ANY),
            pl.BlockSpec(memory_space=pl.ANY),
        ],
        out_specs=pl.BlockSpec(memory_space=pl.ANY),
        scratch_shapes=[
            pltpu.VMEM((m_blk, k_blk), jnp.float32),
            pltpu.VMEM((m_blk, k_blk), jnp.bfloat16),
            pltpu.VMEM((N_DEV - 1, m_blk, k_blk), jnp.bfloat16),
            pltpu.VMEM((N_DEV - 1, m_blk, k_blk), jnp.bfloat16),
            pltpu.VMEM((2, k_blk, N_CHUNK), jnp.float32),
            pltpu.VMEM((m_blk, n_tot), jnp.float32),
            pltpu.SemaphoreType.DMA((N_DEV - 1,)),
            pltpu.SemaphoreType.DMA((N_DEV - 1,)),
            pltpu.SemaphoreType.DMA((2,)),
            pltpu.SemaphoreType.DMA,
            pltpu.SemaphoreType.DMA((2,)),
        ],
        compiler_params=pltpu.CompilerParams(
            collective_id=0,
            vmem_limit_bytes=63 * 1024 * 1024,
        ),
    )(x, w_mat)
